# baseline (speedup 1.0000x reference)
import functools

import jax
import jax.numpy as jnp
from jax import lax
from jax.experimental import pallas as pl
from jax.experimental.pallas import tpu as pltpu
from jax.experimental.pallas import tpu_sc as plsc

B = 128
N = 32768
K = 8
LANES = 16
NUM_CORES = 2
NUM_SUBCORES = 16
NUM_WORKERS = NUM_CORES * NUM_SUBCORES
COLS_PER_WORKER = N // NUM_WORKERS
W = 256
CHUNKS = COLS_PER_WORKER // W
GROUPS = W // LANES
TC_PER_W = W // 128
ROW_BLOCKS = B // K

_SORT8 = ((0, 1), (2, 3), (4, 5), (6, 7),
          (0, 2), (1, 3), (4, 6), (5, 7),
          (1, 2), (5, 6),
          (0, 4), (1, 5), (2, 6), (3, 7),
          (2, 4), (3, 5),
          (1, 2), (3, 4), (5, 6))
_BITONIC8 = ((0, 4), (1, 5), (2, 6), (3, 7),
             (0, 2), (1, 3), (4, 6), (5, 7),
             (0, 1), (2, 3), (4, 5), (6, 7))


def _net_desc(vals, net):
    vals = list(vals)
    for i, j in net:
        hi = jnp.maximum(vals[i], vals[j])
        lo = jnp.minimum(vals[i], vals[j])
        vals[i], vals[j] = hi, lo
    return vals


def _topk_mask_body(a_hbm, out_hbm, buf, buf2, obuf, thr_buf):
    wid = lax.axis_index("s") * NUM_CORES + lax.axis_index("c")
    base = wid * COLS_PER_WORKER
    tc_base = wid * (COLS_PER_WORKER // 128)
    half = B // 2
    zero = jnp.zeros((LANES,), jnp.float32)

    for chunk in range(CHUNKS):
        c0 = base + chunk * W
        pltpu.sync_copy(a_hbm.at[:, pl.ds(c0, W)], buf)

        def group_body(g, _, chunk=chunk):
            col = g * LANES
            tcc = chunk * TC_PER_W + g // (128 // LANES)
            lane0 = (g % (128 // LANES)) * LANES

            run = _net_desc(
                [buf[j, pl.ds(col, LANES)] for j in range(K)], _SORT8)

            def blk_body(rb, run):
                s = _net_desc(
                    [buf[rb * K + j, pl.ds(col, LANES)] for j in range(K)],
                    _SORT8)
                merged = [jnp.maximum(run[i], s[K - 1 - i]) for i in range(K)]
                return tuple(_net_desc(merged, _BITONIC8))

            run = lax.fori_loop(1, ROW_BLOCKS, blk_body, tuple(run))
            thr = run[K - 1]
            thr_buf[pl.ds(chunk * W + col, LANES)] = thr

            def mask_body(rb, _):
                for j in range(K):
                    r = rb * K + j
                    v = buf[r, pl.ds(col, LANES)]
                    obuf[r, tcc, pl.ds(lane0, LANES)] = jnp.where(
                        v >= thr, v, zero)
                return 0

            lax.fori_loop(0, ROW_BLOCKS // 2, mask_body, 0)
            return 0

        lax.fori_loop(0, GROUPS, group_body, 0)

    pltpu.sync_copy(
        obuf, out_hbm.at[pl.ds(0, half), pl.ds(tc_base, COLS_PER_WORKER // 128), :])

    for chunk in range(CHUNKS):
        c0 = base + chunk * W
        pltpu.sync_copy(a_hbm.at[pl.ds(half, half), pl.ds(c0, W)], buf2)

        def group2_body(g, _, chunk=chunk):
            col = g * LANES
            tcc = chunk * TC_PER_W + g // (128 // LANES)
            lane0 = (g % (128 // LANES)) * LANES
            thr = thr_buf[pl.ds(chunk * W + col, LANES)]

            def mask_body(rb, _):
                for j in range(K):
                    r = rb * K + j
                    v = buf2[r, pl.ds(col, LANES)]
                    obuf[r, tcc, pl.ds(lane0, LANES)] = jnp.where(
                        v >= thr, v, zero)
                return 0

            lax.fori_loop(0, ROW_BLOCKS // 2, mask_body, 0)
            return 0

        lax.fori_loop(0, GROUPS, group2_body, 0)

    pltpu.sync_copy(
        obuf,
        out_hbm.at[pl.ds(half, half), pl.ds(tc_base, COLS_PER_WORKER // 128), :])


@jax.jit
def _topk_mask(activations):
    mesh = plsc.VectorSubcoreMesh(core_axis_name="c", subcore_axis_name="s")
    f = functools.partial(
        pl.kernel,
        out_type=jax.ShapeDtypeStruct((B, N // 128, 128), jnp.float32),
        mesh=mesh,
        scratch_types=[pltpu.VMEM((B, W), jnp.float32),
                       pltpu.VMEM((B // 2, W), jnp.float32),
                       pltpu.VMEM((B // 2, COLS_PER_WORKER // 128, 128),
                                  jnp.float32),
                       pltpu.VMEM((COLS_PER_WORKER,), jnp.float32)],
    )(_topk_mask_body)
    return f(activations)


def kernel(activations):
    return _topk_mask(activations).reshape(B, N, 1, 1)


# speedup vs baseline: 1.1145x; 1.1145x over previous
import functools

import jax
import jax.numpy as jnp
from jax import lax
from jax.experimental import pallas as pl
from jax.experimental.pallas import tpu as pltpu
from jax.experimental.pallas import tpu_sc as plsc

B = 128
N = 32768
K = 8
LANES = 16
NUM_CORES = 2
NUM_SUBCORES = 16
NUM_TILES = NUM_CORES * NUM_SUBCORES
COLS_PER_TILE = N // NUM_TILES
TILES_PER_SPAN = COLS_PER_TILE // 128
W = 128
CHUNKS = COLS_PER_TILE // W
GROUPS = W // LANES
ROW_BLOCKS = B // K
BAND1 = 48
BAND3 = 32
BAND2 = B - BAND1 - BAND3

_SORT8 = ((0, 1), (2, 3), (4, 5), (6, 7),
          (0, 2), (1, 3), (4, 6), (5, 7),
          (1, 2), (5, 6),
          (0, 4), (1, 5), (2, 6), (3, 7),
          (2, 4), (3, 5),
          (1, 2), (3, 4), (5, 6))
_BITONIC8 = ((0, 4), (1, 5), (2, 6), (3, 7),
             (0, 2), (1, 3), (4, 6), (5, 7),
             (0, 1), (2, 3), (4, 5), (6, 7))


def _net_desc(vals, net):
    vals = list(vals)
    for i, j in net:
        hi = jnp.maximum(vals[i], vals[j])
        lo = jnp.minimum(vals[i], vals[j])
        vals[i], vals[j] = hi, lo
    return vals


def _topk_mask_body(a_hbm, out_hbm, buf_a, buf_b, obuf_a, obuf_b, thr_buf,
                    sem_a, sem_b, sem_o1, sem_o3):
    cc = lax.axis_index("c")
    sid = lax.axis_index("s")
    tbase = (cc * NUM_SUBCORES + sid) * TILES_PER_SPAN
    cbase = tbase * 128
    zero = jnp.zeros((LANES,), jnp.float32)

    bufs = (buf_a, buf_b)
    in_sems = (sem_a, sem_b)

    def pass1_chunk(ibuf, ch):
        def group_body(g, _):
            col = g * LANES

            run = _net_desc(
                [ibuf[j, pl.ds(col, LANES)] for j in range(K)], _SORT8)

            def blk_body(rb, run):
                s = _net_desc(
                    [ibuf[rb * K + j, pl.ds(col, LANES)] for j in range(K)],
                    _SORT8)
                merged = [jnp.maximum(run[i], s[K - 1 - i]) for i in range(K)]
                return tuple(_net_desc(merged, _BITONIC8))

            run = lax.fori_loop(1, ROW_BLOCKS, blk_body, tuple(run))
            thr = run[K - 1]
            thr_buf[pl.ds(ch * W + col, LANES)] = thr

            def mask1_body(rb, _):
                for j in range(K):
                    r = rb * K + j
                    v = ibuf[r, pl.ds(col, LANES)]
                    obuf_a[r, ch, pl.ds(col, LANES)] = jnp.where(
                        v >= thr, v, zero)
                return 0

            lax.fori_loop(0, BAND1 // K, mask1_body, 0)

            def mask3_body(rb, _):
                for j in range(K):
                    r = (BAND1 + BAND2) + rb * K + j
                    v = ibuf[r, pl.ds(col, LANES)]
                    obuf_b[r - (BAND1 + BAND2), ch, pl.ds(col, LANES)] = (
                        jnp.where(v >= thr, v, zero))
                return 0

            lax.fori_loop(0, BAND3 // K, mask3_body, 0)
            return 0

        lax.fori_loop(0, GROUPS, group_body, 0)

    def pass2_chunk(ibuf, ch):
        def group_body(g, _):
            col = g * LANES
            thr = thr_buf[pl.ds(ch * W + col, LANES)]

            def mask_body(rb, _):
                for j in range(K):
                    r = rb * K + j
                    v = ibuf[r, pl.ds(col, LANES)]
                    obuf_a[r, ch, pl.ds(col, LANES)] = jnp.where(
                        v >= thr, v, zero)
                return 0

            lax.fori_loop(0, BAND2 // K, mask_body, 0)
            return 0

        lax.fori_loop(0, GROUPS, group_body, 0)

    in_h = [None] * CHUNKS
    in_h[0] = pltpu.async_copy(
        a_hbm.at[:, pl.ds(cbase, W)], bufs[0], in_sems[0])
    for ch in range(CHUNKS):
        nxt = ch + 1
        if nxt < CHUNKS:
            in_h[nxt] = pltpu.async_copy(
                a_hbm.at[:, pl.ds(cbase + nxt * W, W)],
                bufs[nxt % 2], in_sems[nxt % 2])
        in_h[ch].wait()
        pass1_chunk(bufs[ch % 2], ch)

    out1_h = pltpu.async_copy(
        obuf_a,
        out_hbm.at[pl.ds(0, BAND1), pl.ds(tbase, TILES_PER_SPAN), :],
        sem_o1)
    out3_h = pltpu.async_copy(
        obuf_b,
        out_hbm.at[pl.ds(BAND1 + BAND2, BAND3),
                   pl.ds(tbase, TILES_PER_SPAN), :],
        sem_o3)

    in2_h = [None] * CHUNKS
    in2_h[0] = pltpu.async_copy(
        a_hbm.at[pl.ds(BAND1, BAND2), pl.ds(cbase, W)],
        bufs[0].at[pl.ds(0, BAND2), :], in_sems[0])
    for ch in range(CHUNKS):
        nxt = ch + 1
        if nxt < CHUNKS:
            in2_h[nxt] = pltpu.async_copy(
                a_hbm.at[pl.ds(BAND1, BAND2), pl.ds(cbase + nxt * W, W)],
                bufs[nxt % 2].at[pl.ds(0, BAND2), :], in_sems[nxt % 2])
        in2_h[ch].wait()
        if ch == 0:
            out1_h.wait()
        pass2_chunk(bufs[ch % 2], ch)

    pltpu.sync_copy(
        obuf_a.at[pl.ds(0, BAND2)],
        out_hbm.at[pl.ds(BAND1, BAND2), pl.ds(tbase, TILES_PER_SPAN), :])
    out3_h.wait()


@jax.jit
def _topk_mask(activations):
    mesh = plsc.VectorSubcoreMesh(core_axis_name="c", subcore_axis_name="s")
    f = functools.partial(
        pl.kernel,
        out_type=jax.ShapeDtypeStruct((B, N // 128, 128), jnp.float32),
        mesh=mesh,
        scratch_types=[
            pltpu.VMEM((B, W), jnp.float32),
            pltpu.VMEM((B, W), jnp.float32),
            pltpu.VMEM((BAND1, TILES_PER_SPAN, 128), jnp.float32),
            pltpu.VMEM((BAND3, TILES_PER_SPAN, 128), jnp.float32),
            pltpu.VMEM((COLS_PER_TILE,), jnp.float32),
            pltpu.SemaphoreType.DMA,
            pltpu.SemaphoreType.DMA,
            pltpu.SemaphoreType.DMA,
            pltpu.SemaphoreType.DMA,
        ],
    )(_topk_mask_body)
    return f(activations)


def kernel(activations):
    return _topk_mask(activations).reshape(B, N, 1, 1)


# speedup vs baseline: 1.4985x; 1.3445x over previous
import functools

import jax
import jax.numpy as jnp
from jax import lax
from jax.experimental import pallas as pl
from jax.experimental.pallas import tpu as pltpu
from jax.experimental.pallas import tpu_sc as plsc

B = 128
N = 32768
K = 8
LANES = 16
NUM_CORES = 2
NUM_SUBCORES = 16
NUM_TILES = NUM_CORES * NUM_SUBCORES
COLS_PER_TILE = N // NUM_TILES
W = 256
CHUNKS = COLS_PER_TILE // W
NBUF = 3
GROUPS = W // LANES
ROW_BLOCKS = B // K

_SORT8 = ((0, 1), (2, 3), (4, 5), (6, 7),
          (0, 2), (1, 3), (4, 6), (5, 7),
          (1, 2), (5, 6),
          (0, 4), (1, 5), (2, 6), (3, 7),
          (2, 4), (3, 5),
          (1, 2), (3, 4), (5, 6))
_BITONIC8 = ((0, 4), (1, 5), (2, 6), (3, 7),
             (0, 2), (1, 3), (4, 6), (5, 7),
             (0, 1), (2, 3), (4, 5), (6, 7))


def _net_desc(vals, net):
    vals = list(vals)
    for i, j in net:
        hi = jnp.maximum(vals[i], vals[j])
        lo = jnp.minimum(vals[i], vals[j])
        vals[i], vals[j] = hi, lo
    return vals


def _topk_mask_body(a_hbm, out_hbm, buf_0, buf_1, buf_2,
                    sem_i0, sem_i1, sem_i2, sem_o0, sem_o1, sem_o2):
    cc = lax.axis_index("c")
    sid = lax.axis_index("s")
    cbase = (cc * NUM_SUBCORES + sid) * COLS_PER_TILE
    zero = jnp.zeros((LANES,), jnp.float32)

    bufs = (buf_0, buf_1, buf_2)
    in_sems = (sem_i0, sem_i1, sem_i2)
    out_sems = (sem_o0, sem_o1, sem_o2)

    def compute_chunk(ibuf):
        def group_body(g, _):
            col = g * LANES

            run = _net_desc(
                [ibuf[j, pl.ds(col, LANES)] for j in range(K)], _SORT8)

            def blk_body(rb, run):
                s = _net_desc(
                    [ibuf[rb * K + j, pl.ds(col, LANES)] for j in range(K)],
                    _SORT8)
                merged = [jnp.maximum(run[i], s[K - 1 - i]) for i in range(K)]
                return tuple(_net_desc(merged, _BITONIC8))

            run = lax.fori_loop(1, ROW_BLOCKS, blk_body, tuple(run))
            thr = run[K - 1]

            def mask_body(rb, _):
                for j in range(K):
                    r = rb * K + j
                    v = ibuf[r, pl.ds(col, LANES)]
                    ibuf[r, pl.ds(col, LANES)] = jnp.where(v >= thr, v, zero)
                return 0

            lax.fori_loop(0, ROW_BLOCKS, mask_body, 0)
            return 0

        lax.fori_loop(0, GROUPS, group_body, 0)

    in_h = [None] * CHUNKS
    out_h = [None] * CHUNKS
    for c in range(min(NBUF, CHUNKS)):
        in_h[c] = pltpu.async_copy(
            a_hbm.at[:, pl.ds(cbase + c * W, W)], bufs[c % NBUF],
            in_sems[c % NBUF])
    waited = set()
    for c in range(CHUNKS):
        in_h[c].wait()
        compute_chunk(bufs[c % NBUF])
        out_h[c] = pltpu.async_copy(
            bufs[c % NBUF], out_hbm.at[:, pl.ds(cbase + c * W, W)],
            out_sems[c % NBUF])
        nxt = c + NBUF - 1
        if NBUF <= nxt < CHUNKS:
            out_h[nxt - NBUF].wait()
            waited.add(nxt - NBUF)
            in_h[nxt] = pltpu.async_copy(
                a_hbm.at[:, pl.ds(cbase + nxt * W, W)], bufs[nxt % NBUF],
                in_sems[nxt % NBUF])
    for c in range(CHUNKS):
        if c not in waited:
            out_h[c].wait()


@jax.jit
def _topk_mask(activations):
    mesh = plsc.VectorSubcoreMesh(core_axis_name="c", subcore_axis_name="s")
    f = functools.partial(
        pl.kernel,
        out_type=jax.ShapeDtypeStruct((B, N), jnp.float32),
        mesh=mesh,
        scratch_types=[
            pltpu.VMEM((B, W), jnp.float32),
            pltpu.VMEM((B, W), jnp.float32),
            pltpu.VMEM((B, W), jnp.float32),
            pltpu.SemaphoreType.DMA,
            pltpu.SemaphoreType.DMA,
            pltpu.SemaphoreType.DMA,
            pltpu.SemaphoreType.DMA,
            pltpu.SemaphoreType.DMA,
            pltpu.SemaphoreType.DMA,
        ],
    )(_topk_mask_body)
    return f(activations)


def kernel(activations):
    return _topk_mask(activations)[:, :, None, None]
